# Initial kernel scaffold; baseline (speedup 1.0000x reference)
import jax
import jax.numpy as jnp
from jax.experimental import pallas as pl


def kernel(x, table):
    raise NotImplementedError("write your pallas kernel here")


# speedup vs baseline: 1.1817x; 1.1817x over previous
import functools

import numpy as np
import jax
import jax.numpy as jnp
from jax import lax
from jax.experimental import pallas as pl
from jax.experimental.pallas import tpu as pltpu
from jax.experimental.pallas import tpu_sc as plsc

VOCAB = 100000
D = 768
B = 4
S = 4096

NC = 2
NS = 16
NW = NC * NS
SW = S // NW
CS = 8
CR = CS * B
NCHUNK = SW // CS
TPW = SW * B
NBUF = 4
PBUF = 3
LANES = 16
KSTEPS = D // LANES


def _pos_encoding(d_model, max_len):
    pos = np.arange(max_len, dtype=np.float32)[:, None]
    _2i = np.arange(0, d_model, 2, dtype=np.float32)
    enc = np.zeros((max_len, d_model), dtype=np.float32)
    enc[:, 0::2] = np.sin(pos / 10000.0 ** (_2i / d_model))
    enc[:, 1::2] = np.cos(pos / 10000.0 ** (_2i / d_model))
    return enc


_POS = _pos_encoding(D, S)

_w = np.arange(NW)[:, None, None]
_i = np.arange(NCHUNK)[None, :, None]
_j = np.arange(CR)[None, None, :]
_OIDX = ((_j % B) * S + _w * SW + _i * CS + _j // B).astype(np.int32)


def _emb_body(xr_hbm, table_hbm, pos_hbm, oidx_hbm, out_hbm,
              idx_v, oidx_v, rows_v, pos_v,
              g0, g1, g2, g3, p0, p1, p2, o0, o1, o2, o3):
    gsems = (g0, g1, g2, g3)
    psems = (p0, p1, p2)
    osems = (o0, o1, o2, o3)

    cid = lax.axis_index("c")
    sid = lax.axis_index("s")
    wid = sid * NC + cid
    s0 = wid * SW

    pltpu.sync_copy(xr_hbm.at[wid], idx_v)
    pltpu.sync_copy(oidx_hbm.at[wid], oidx_v)

    def start_in(i):
        q = i % NBUF
        p = i % PBUF
        g = pltpu.async_copy(table_hbm.at[idx_v.at[i]], rows_v.at[q], gsems[q])
        pp = pltpu.async_copy(pos_hbm.at[pl.ds(s0 + i * CS, CS)], pos_v.at[p],
                              psems[p])
        return g, pp

    ins = {0: start_in(0), 1: start_in(1)}
    outs = {}
    for i in range(NCHUNK):
        q = i % NBUF
        p = i % PBUF
        if i + 2 < NCHUNK:
            if i - 2 >= 0:
                outs[i - 2].wait()
            ins[i + 2] = start_in(i + 2)
        g, pp = ins.pop(i)
        g.wait()
        pp.wait()
        for sl in range(CS):
            def kbody(k, carry, _q=q, _p=p, _sl=sl):
                off = pl.multiple_of(k * LANES, LANES)
                pv = pos_v[_p, _sl, pl.ds(off, LANES)]
                for bb in range(B):
                    plsc.addupdate(rows_v.at[_q, _sl * B + bb, pl.ds(off, LANES)],
                                   pv)
                return carry
            lax.fori_loop(0, KSTEPS, kbody, 0)
        outs[i] = pltpu.async_copy(rows_v.at[q], out_hbm.at[oidx_v.at[i]],
                                   osems[q])
    for i in range(NCHUNK - NBUF, NCHUNK):
        outs[i].wait()


@functools.cache
def _emb():
    return pl.kernel(
        _emb_body,
        out_type=jax.ShapeDtypeStruct((B * S, D), jnp.float32),
        mesh=plsc.VectorSubcoreMesh(core_axis_name="c", subcore_axis_name="s",
                                    num_cores=NC, num_subcores=NS),
        scratch_types=[
            pltpu.VMEM((NCHUNK, CR), jnp.int32),
            pltpu.VMEM((NCHUNK, CR), jnp.int32),
            pltpu.VMEM((NBUF, CR, D), jnp.float32),
            pltpu.VMEM((PBUF, CS, D), jnp.float32),
        ] + [pltpu.SemaphoreType.DMA] * (NBUF + PBUF + NBUF),
    )


def kernel(x, table):
    xr = jnp.swapaxes(x.astype(jnp.int32), 0, 1).reshape(NW, NCHUNK, CR)
    out = _emb()(xr, table, jnp.asarray(_POS), jnp.asarray(_OIDX))
    return out.reshape(B, S, D)


# speedup vs baseline: 1.1979x; 1.0137x over previous
import functools

import numpy as np
import jax
import jax.numpy as jnp
from jax import lax
from jax.experimental import pallas as pl
from jax.experimental.pallas import tpu as pltpu
from jax.experimental.pallas import tpu_sc as plsc

VOCAB = 100000
D = 768
B = 4
S = 4096

NC = 2
NS = 16
NW = NC * NS
SW = S // NW
CS = 8
CR = CS * B
NCHUNK = SW // CS
TPW = SW * B
NBUF = 4
PBUF = 3
LANES = 16
KSTEPS = D // LANES


def _pos_encoding(d_model, max_len):
    pos = np.arange(max_len, dtype=np.float32)[:, None]
    _2i = np.arange(0, d_model, 2, dtype=np.float32)
    enc = np.zeros((max_len, d_model), dtype=np.float32)
    enc[:, 0::2] = np.sin(pos / 10000.0 ** (_2i / d_model))
    enc[:, 1::2] = np.cos(pos / 10000.0 ** (_2i / d_model))
    return enc


_POS = _pos_encoding(D, S)

_w = np.arange(NW)[:, None, None]
_i = np.arange(NCHUNK)[None, :, None]
_j = np.arange(CR)[None, None, :]
_OIDX = ((_j % B) * S + _w * SW + _i * CS + _j // B).astype(np.int32)


def _emb_body(xr_hbm, table_hbm, pos_hbm, oidx_hbm, out_hbm,
              idx_v, oidx_v, rows_v, pos_v,
              g0, g1, g2, g3, p0, p1, p2, o0, o1, o2, o3):
    gsems = (g0, g1, g2, g3)
    psems = (p0, p1, p2)
    osems = (o0, o1, o2, o3)

    cid = lax.axis_index("c")
    sid = lax.axis_index("s")
    wid = sid * NC + cid
    s0 = wid * SW

    pltpu.sync_copy(xr_hbm.at[wid], idx_v)
    pltpu.sync_copy(oidx_hbm.at[wid], oidx_v)

    def start_in(i):
        q = i % NBUF
        p = i % PBUF
        g = pltpu.async_copy(table_hbm.at[idx_v.at[i]], rows_v.at[q], gsems[q])
        pp = pltpu.async_copy(pos_hbm.at[pl.ds(s0 + i * CS, CS)], pos_v.at[p],
                              psems[p])
        return g, pp

    ins = {0: start_in(0), 1: start_in(1)}
    outs = {}
    for i in range(NCHUNK):
        q = i % NBUF
        p = i % PBUF
        if i + 2 < NCHUNK:
            if i - 2 >= 0:
                outs[i - 2].wait()
            ins[i + 2] = start_in(i + 2)
        g, pp = ins.pop(i)
        g.wait()
        pp.wait()
        for sl in range(CS):
            @plsc.parallel_loop(0, KSTEPS, unroll=2)
            def _add(k, _q=q, _p=p, _sl=sl):
                off = pl.multiple_of(k * LANES, LANES)
                pv = pos_v[_p, _sl, pl.ds(off, LANES)]
                for bb in range(B):
                    plsc.addupdate(rows_v.at[_q, _sl * B + bb, pl.ds(off, LANES)],
                                   pv)
        outs[i] = pltpu.async_copy(rows_v.at[q], out_hbm.at[oidx_v.at[i]],
                                   osems[q])
    for i in range(NCHUNK - NBUF, NCHUNK):
        outs[i].wait()


@functools.cache
def _emb():
    return pl.kernel(
        _emb_body,
        out_type=jax.ShapeDtypeStruct((B * S, D), jnp.float32),
        mesh=plsc.VectorSubcoreMesh(core_axis_name="c", subcore_axis_name="s",
                                    num_cores=NC, num_subcores=NS),
        scratch_types=[
            pltpu.VMEM((NCHUNK, CR), jnp.int32),
            pltpu.VMEM((NCHUNK, CR), jnp.int32),
            pltpu.VMEM((NBUF, CR, D), jnp.float32),
            pltpu.VMEM((PBUF, CS, D), jnp.float32),
        ] + [pltpu.SemaphoreType.DMA] * (NBUF + PBUF + NBUF),
    )


def kernel(x, table):
    xr = jnp.swapaxes(x.astype(jnp.int32), 0, 1).reshape(NW, NCHUNK, CR)
    out = _emb()(xr, table, jnp.asarray(_POS), jnp.asarray(_OIDX))
    return out.reshape(B, S, D)


# speedup vs baseline: 1.2575x; 1.0497x over previous
import functools

import numpy as np
import jax
import jax.numpy as jnp
from jax import lax
from jax.experimental import pallas as pl
from jax.experimental.pallas import tpu as pltpu
from jax.experimental.pallas import tpu_sc as plsc

VOCAB = 100000
D = 768
B = 4
S = 4096

NC = 2
NS = 16
NW = NC * NS
SW = S // NW
CS = 8
NCHUNK = SW // CS
NBUF = 4
PBUF = 3
LANES = 16
KSTEPS = D // LANES


def _pos_encoding(d_model, max_len):
    pos = np.arange(max_len, dtype=np.float32)[:, None]
    _2i = np.arange(0, d_model, 2, dtype=np.float32)
    enc = np.zeros((max_len, d_model), dtype=np.float32)
    enc[:, 0::2] = np.sin(pos / 10000.0 ** (_2i / d_model))
    enc[:, 1::2] = np.cos(pos / 10000.0 ** (_2i / d_model))
    return enc


_POS = _pos_encoding(D, S)


def _emb_body(x_hbm, table_hbm, pos_hbm, out_hbm,
              idx_v, rows_v, pos_v,
              g0, g1, g2, g3, p0, p1, p2, o0, o1, o2, o3):
    gsems = (g0, g1, g2, g3)
    psems = (p0, p1, p2)
    osems = (o0, o1, o2, o3)

    cid = lax.axis_index("c")
    sid = lax.axis_index("s")
    wid = sid * NC + cid
    s0 = wid * SW

    for bb in range(B):
        pltpu.sync_copy(x_hbm.at[bb, pl.ds(s0, SW)], idx_v.at[bb])

    def start_in(i):
        q = i % NBUF
        p = i % PBUF
        gs = tuple(
            pltpu.async_copy(table_hbm.at[idx_v.at[bb, pl.ds(i * CS, CS)]],
                             rows_v.at[q, bb], gsems[q])
            for bb in range(B))
        pp = pltpu.async_copy(pos_hbm.at[pl.ds(s0 + i * CS, CS)], pos_v.at[p],
                              psems[p])
        return gs, pp

    ins = {0: start_in(0), 1: start_in(1)}
    outs = {}
    for i in range(NCHUNK):
        q = i % NBUF
        p = i % PBUF
        if i + 2 < NCHUNK:
            if i - 2 >= 0:
                for o in outs.pop(i - 2):
                    o.wait()
            ins[i + 2] = start_in(i + 2)
        gs, pp = ins.pop(i)
        for g in gs:
            g.wait()
        pp.wait()
        for sl in range(CS):
            @plsc.parallel_loop(0, KSTEPS, unroll=2)
            def _add(k, _q=q, _p=p, _sl=sl):
                off = pl.multiple_of(k * LANES, LANES)
                pv = pos_v[_p, _sl, pl.ds(off, LANES)]
                for bb in range(B):
                    plsc.addupdate(rows_v.at[_q, bb, _sl, pl.ds(off, LANES)],
                                   pv)
        outs[i] = tuple(
            pltpu.async_copy(rows_v.at[q, bb],
                             out_hbm.at[bb, pl.ds(s0 + i * CS, CS)], osems[q])
            for bb in range(B))
    for i in range(NCHUNK - NBUF, NCHUNK):
        for o in outs.pop(i):
            o.wait()


@functools.cache
def _emb():
    return pl.kernel(
        _emb_body,
        out_type=jax.ShapeDtypeStruct((B, S, D), jnp.float32),
        mesh=plsc.VectorSubcoreMesh(core_axis_name="c", subcore_axis_name="s",
                                    num_cores=NC, num_subcores=NS),
        scratch_types=[
            pltpu.VMEM((B, SW), jnp.int32),
            pltpu.VMEM((NBUF, B, CS, D), jnp.float32),
            pltpu.VMEM((PBUF, CS, D), jnp.float32),
        ] + [pltpu.SemaphoreType.DMA] * (NBUF + PBUF + NBUF),
    )


def kernel(x, table):
    return _emb()(x.astype(jnp.int32), table, jnp.asarray(_POS))
